# baseline (speedup 1.0000x reference)
import functools
import math

import jax
import jax.numpy as jnp
from jax import lax
from jax.experimental import pallas as pl
from jax.experimental.pallas import tpu as pltpu
from jax.experimental.pallas import tpu_sc as plsc

D_MODEL = 64
SCALE = math.sqrt(D_MODEL)
HB = 20
NBLK = 10
VC = 7808


def _detile_lut(lutt, vmain):
    d, vocab = lutt.shape
    info = plsc.get_sparse_core_info()
    nc, ns = info.num_cores, info.num_subcores
    nw = nc * ns
    nq = nw // (d // 8)
    vpw = vmain // nq
    nck = -(-vpw // VC)
    assert vpw % 128 == 0 and vpw * nq == vmain

    mesh = plsc.VectorSubcoreMesh(core_axis_name="c", subcore_axis_name="s")

    @functools.partial(
        pl.kernel,
        mesh=mesh,
        out_type=jax.ShapeDtypeStruct((d * vmain,), jnp.float32),
        scratch_types=[
            pltpu.VMEM((2, 8, VC), jnp.float32),
            pltpu.SemaphoreType.DMA,
            pltpu.SemaphoreType.DMA,
            pltpu.SemaphoreType.DMA,
            pltpu.SemaphoreType.DMA,
        ],
    )
    def detile(lutt_hbm, out_hbm, vb, r0, r1, w0, w1):
        rs = (r0, r1)
        ws = (w0, w1)
        cid = lax.axis_index("c")
        sid = lax.axis_index("s")
        w = sid * nc + cid
        a = w // nq
        v0 = (w % nq) * vpw

        def csize(i):
            return min(VC, vpw - i * VC)

        def fire_read(i):
            n = csize(i)
            pltpu.async_copy(
                lutt_hbm.at[pl.ds(a * 8, 8), pl.ds(v0 + i * VC, n)],
                vb.at[i % 2, :, pl.ds(0, n)], rs[i % 2])

        def wait_read(i):
            n = csize(i)
            pltpu.make_async_copy(
                lutt_hbm.at[pl.ds(0, 8), pl.ds(0, n)],
                vb.at[i % 2, :, pl.ds(0, n)], rs[i % 2]).wait()

        def fire_writes(i):
            n = csize(i)
            for s in range(8):
                pltpu.async_copy(
                    vb.at[i % 2, s, pl.ds(0, n)],
                    out_hbm.at[pl.ds((a * 8 + s) * vmain + v0 + i * VC, n)],
                    ws[i % 2])

        def wait_writes(i):
            n = csize(i)
            for s in range(8):
                pltpu.make_async_copy(
                    vb.at[i % 2, s, pl.ds(0, n)],
                    out_hbm.at[pl.ds(0, n)], ws[i % 2]).wait()

        fire_read(0)
        for i in range(nck):
            wait_read(i)
            if i + 1 < nck:
                if i >= 1:
                    wait_writes(i - 1)
                fire_read(i + 1)
            fire_writes(i)
        if nck >= 2:
            wait_writes(nck - 2)
        wait_writes(nck - 1)

    return detile(lutt)


def kernel(x, lut):
    batch, hist = x.shape
    vocab, d = lut.shape
    info = plsc.get_sparse_core_info()
    nc, ns = info.num_cores, info.num_subcores
    fps = d // nc
    bpt = batch // ns
    blk = HB * bpt
    npt = hist * bpt
    vmain = (vocab // 128) * 128
    vtail = vocab - vmain
    spt = vmain // ns
    assert fps * nc == d and bpt * ns == batch and NBLK * HB == hist
    assert spt * ns == vmain and spt % 8 == 0 and bpt == 256

    lutt = jnp.transpose(lut)
    xt = jnp.transpose(x).astype(jnp.int32)
    tailt = lutt[:, vmain:]
    scr = _detile_lut(lutt, vmain)

    mesh = plsc.VectorSubcoreMesh(core_axis_name="c", subcore_axis_name="s")

    @functools.partial(
        pl.kernel,
        mesh=mesh,
        out_type=jax.ShapeDtypeStruct((hist, d // 8, batch // 128, 8, 128),
                                      jnp.float32),
        scratch_types=[
            pltpu.VMEM((npt,), jnp.int32),
            pltpu.VMEM((2, blk), jnp.float32),
            pltpu.VMEM((HB, 2, 128), jnp.float32),
            pltpu.VMEM_SHARED((vocab,), jnp.float32),
            pltpu.SemaphoreType.DMA,
            pltpu.SemaphoreType.DMA,
            pltpu.SemaphoreType.DMA,
            pltpu.SemaphoreType.DMA,
        ],
        compiler_params=pltpu.CompilerParams(use_tc_tiling_on_sc=False),
    )
    def emb(scr_hbm, xt_hbm, tail_hbm, out_hbm, idx_v, gb, wb, feat_sh,
            gs0, gs1, wsem, ssem):
        gs = (gs0, gs1)
        cid = lax.axis_index("c")
        sid = lax.axis_index("s")
        b0 = sid * bpt

        for h in range(hist):
            pltpu.async_copy(
                xt_hbm.at[h, pl.ds(b0, bpt)],
                idx_v.at[pl.ds(h * bpt, bpt)], ssem)
        for h in range(hist):
            pltpu.make_async_copy(
                xt_hbm.at[h, pl.ds(b0, bpt)],
                idx_v.at[pl.ds(h * bpt, bpt)], ssem).wait()

        def fire_stage(f):
            pltpu.async_copy(
                scr_hbm.at[pl.ds(f * vmain + sid * spt, spt)],
                feat_sh.at[pl.ds(sid * spt, spt)], ssem)

            @pl.when(sid == 0)
            def _():
                pltpu.async_copy(
                    tail_hbm.at[f], feat_sh.at[pl.ds(vmain, vtail)], ssem)

        def wait_stage():
            pltpu.make_async_copy(
                scr_hbm.at[pl.ds(0, spt)],
                feat_sh.at[pl.ds(0, spt)], ssem).wait()

            @pl.when(sid == 0)
            def _():
                pltpu.make_async_copy(
                    tail_hbm.at[0], feat_sh.at[pl.ds(vmain, vtail)],
                    ssem).wait()

        def fire_gather(k):
            pltpu.async_copy(
                feat_sh.at[idx_v.at[pl.ds(k * blk, blk)]],
                gb.at[k % 2], gs[k % 2])

        def wait_gather(k):
            pltpu.make_async_copy(
                feat_sh.at[idx_v.at[pl.ds(0, blk)]],
                gb.at[k % 2], gs[k % 2]).wait()

        def fire_write(f, k):
            a = f // 8
            s = f % 8
            pltpu.async_copy(
                wb,
                out_hbm.at[pl.ds(k * HB, HB), a, pl.ds(2 * sid, 2), s],
                wsem)

        def wait_write():
            pltpu.make_async_copy(
                wb,
                out_hbm.at[pl.ds(0, HB), 0, pl.ds(2 * sid, 2), 0],
                wsem).wait()

        def scale_block(k):
            src = gb.at[k % 2]
            dst = wb

            @pl.loop(0, HB, unroll=4)
            def scale_row(h):
                for c in range(2):
                    for g in range(8):
                        dst[h, c, pl.ds(g * 16, 16)] = (
                            src[pl.ds(h * bpt + c * 128 + g * 16, 16)]
                            * SCALE)

        fire_stage(cid * fps)
        wait_stage()
        plsc.subcore_barrier()

        @pl.loop(0, fps)
        def feat_loop(fi):
            f = cid * fps + fi
            fire_gather(0)
            fire_gather(1)
            for k in range(NBLK):
                wait_gather(k)
                if k == NBLK - 1:
                    plsc.subcore_barrier()

                    @pl.when(fi < fps - 1)
                    def _():
                        fire_stage(f + 1)
                if k >= 1:
                    wait_write()
                else:
                    @pl.when(fi > 0)
                    def _():
                        wait_write()
                scale_block(k)
                fire_write(f, k)
                if k + 2 < NBLK:
                    fire_gather(k + 2)

            @pl.when(fi < fps - 1)
            def _():
                wait_stage()

            plsc.subcore_barrier()

        wait_write()

    out5 = emb(scr, xt, tailt)
    return jnp.transpose(out5, (2, 4, 0, 1, 3)).reshape(batch, hist, d)


# speedup vs baseline: 1.1318x; 1.1318x over previous
import functools
import math

import jax
import jax.numpy as jnp
from jax import lax
from jax.experimental import pallas as pl
from jax.experimental.pallas import tpu as pltpu
from jax.experimental.pallas import tpu_sc as plsc

D_MODEL = 64
SCALE = math.sqrt(D_MODEL)
HB = 20
NBLK = 10
VC = 7808


def _detile_lut(lutt, vmain):
    d, vocab = lutt.shape
    info = plsc.get_sparse_core_info()
    nc, ns = info.num_cores, info.num_subcores
    nw = nc * ns
    nq = nw // (d // 8)
    vpw = vmain // nq
    nck = -(-vpw // VC)
    assert vpw % 128 == 0 and vpw * nq == vmain

    mesh = plsc.VectorSubcoreMesh(core_axis_name="c", subcore_axis_name="s")

    @functools.partial(
        pl.kernel,
        mesh=mesh,
        out_type=jax.ShapeDtypeStruct((d * vmain,), jnp.float32),
        scratch_types=[
            pltpu.VMEM((2, 8, VC), jnp.float32),
            pltpu.SemaphoreType.DMA,
            pltpu.SemaphoreType.DMA,
            pltpu.SemaphoreType.DMA,
            pltpu.SemaphoreType.DMA,
        ],
    )
    def detile(lutt_hbm, out_hbm, vb, r0, r1, w0, w1):
        rs = (r0, r1)
        ws = (w0, w1)
        cid = lax.axis_index("c")
        sid = lax.axis_index("s")
        w = sid * nc + cid
        a = w // nq
        v0 = (w % nq) * vpw

        def csize(i):
            return min(VC, vpw - i * VC)

        def fire_read(i):
            n = csize(i)
            pltpu.async_copy(
                lutt_hbm.at[pl.ds(a * 8, 8), pl.ds(v0 + i * VC, n)],
                vb.at[i % 2, :, pl.ds(0, n)], rs[i % 2])

        def wait_read(i):
            n = csize(i)
            pltpu.make_async_copy(
                lutt_hbm.at[pl.ds(0, 8), pl.ds(0, n)],
                vb.at[i % 2, :, pl.ds(0, n)], rs[i % 2]).wait()

        def fire_writes(i):
            n = csize(i)
            for s in range(8):
                pltpu.async_copy(
                    vb.at[i % 2, s, pl.ds(0, n)],
                    out_hbm.at[pl.ds((a * 8 + s) * vmain + v0 + i * VC, n)],
                    ws[i % 2])

        def wait_writes(i):
            n = csize(i)
            for s in range(8):
                pltpu.make_async_copy(
                    vb.at[i % 2, s, pl.ds(0, n)],
                    out_hbm.at[pl.ds(0, n)], ws[i % 2]).wait()

        fire_read(0)
        for i in range(nck):
            wait_read(i)
            if i + 1 < nck:
                if i >= 1:
                    wait_writes(i - 1)
                fire_read(i + 1)
            fire_writes(i)
        if nck >= 2:
            wait_writes(nck - 2)
        wait_writes(nck - 1)

    return detile(lutt)


def kernel(x, lut):
    batch, hist = x.shape
    vocab, d = lut.shape
    info = plsc.get_sparse_core_info()
    nc, ns = info.num_cores, info.num_subcores
    fps = d // nc
    bpt = batch // ns
    blk = HB * bpt
    npt = hist * bpt
    vmain = (vocab // 128) * 128
    vtail = vocab - vmain
    spt = vmain // ns
    assert fps * nc == d and bpt * ns == batch and NBLK * HB == hist
    assert spt * ns == vmain and spt % 8 == 0 and bpt == 256

    lutt = jnp.transpose(lut)
    xt = jnp.transpose(x).astype(jnp.int32)
    tailt = lutt[:, vmain:]
    scr = _detile_lut(lutt, vmain)

    mesh = plsc.VectorSubcoreMesh(core_axis_name="c", subcore_axis_name="s")

    @functools.partial(
        pl.kernel,
        mesh=mesh,
        out_type=jax.ShapeDtypeStruct((hist, d // 8, batch // 128, 8, 128),
                                      jnp.float32),
        scratch_types=[
            pltpu.VMEM((npt,), jnp.int32),
            pltpu.VMEM((2, blk), jnp.float32),
            pltpu.VMEM((HB, 2, 128), jnp.float32),
            pltpu.VMEM_SHARED((vocab,), jnp.float32),
            pltpu.SemaphoreType.DMA,
            pltpu.SemaphoreType.DMA,
            pltpu.SemaphoreType.DMA,
            pltpu.SemaphoreType.DMA,
        ],
        compiler_params=pltpu.CompilerParams(use_tc_tiling_on_sc=False),
    )
    def emb(scr_hbm, xt_hbm, tail_hbm, out_hbm, idx_v, gb, wb, feat_sh,
            gs0, gs1, wsem, ssem):
        gs = (gs0, gs1)
        cid = lax.axis_index("c")
        sid = lax.axis_index("s")
        b0 = sid * bpt

        for h in range(hist):
            pltpu.async_copy(
                xt_hbm.at[h, pl.ds(b0, bpt)],
                idx_v.at[pl.ds(h * bpt, bpt)], ssem)
        for h in range(hist):
            pltpu.make_async_copy(
                xt_hbm.at[h, pl.ds(b0, bpt)],
                idx_v.at[pl.ds(h * bpt, bpt)], ssem).wait()

        def fire_stage(f):
            pltpu.async_copy(
                scr_hbm.at[pl.ds(f * vmain + sid * spt, spt)],
                feat_sh.at[pl.ds(sid * spt, spt)], ssem)

            @pl.when(sid == 0)
            def _():
                pltpu.async_copy(
                    tail_hbm.at[f], feat_sh.at[pl.ds(vmain, vtail)], ssem)

        def wait_stage():
            pltpu.make_async_copy(
                scr_hbm.at[pl.ds(0, spt)],
                feat_sh.at[pl.ds(0, spt)], ssem).wait()

            @pl.when(sid == 0)
            def _():
                pltpu.make_async_copy(
                    tail_hbm.at[0], feat_sh.at[pl.ds(vmain, vtail)],
                    ssem).wait()

        def fire_gather(k):
            pltpu.async_copy(
                feat_sh.at[idx_v.at[pl.ds(k * blk, blk)]],
                gb.at[k % 2], gs[k % 2])

        def wait_gather(k):
            pltpu.make_async_copy(
                feat_sh.at[idx_v.at[pl.ds(0, blk)]],
                gb.at[k % 2], gs[k % 2]).wait()

        def fire_write(f, k):
            a = f // 8
            s = f % 8
            pltpu.async_copy(
                wb,
                out_hbm.at[pl.ds(k * HB, HB), a, pl.ds(2 * sid, 2), s],
                wsem)

        def wait_write():
            pltpu.make_async_copy(
                wb,
                out_hbm.at[pl.ds(0, HB), 0, pl.ds(2 * sid, 2), 0],
                wsem).wait()

        def scale_block(k):
            src = gb.at[k % 2]
            dst = wb

            @pl.loop(0, HB)
            def scale_row(h):
                for c in range(2):
                    for g in range(8):
                        dst[h, c, pl.ds(g * 16, 16)] = (
                            src[pl.ds(h * bpt + c * 128 + g * 16, 16)]
                            * SCALE)

        fire_stage(cid * fps)
        wait_stage()
        plsc.subcore_barrier()

        @pl.loop(0, fps)
        def feat_loop(fi):
            f = cid * fps + fi
            fire_gather(0)
            fire_gather(1)
            for k in range(NBLK):
                wait_gather(k)
                if k == NBLK - 1:
                    plsc.subcore_barrier()

                    @pl.when(fi < fps - 1)
                    def _():
                        fire_stage(f + 1)
                if k >= 1:
                    wait_write()
                else:
                    @pl.when(fi > 0)
                    def _():
                        wait_write()
                scale_block(k)
                fire_write(f, k)
                if k + 2 < NBLK:
                    fire_gather(k + 2)

            @pl.when(fi < fps - 1)
            def _():
                wait_stage()

            plsc.subcore_barrier()

        wait_write()

    out5 = emb(scr, xt, tailt)
    return jnp.transpose(out5, (2, 4, 0, 1, 3)).reshape(batch, hist, d)
